# baseline (device time: 122869 ns/iter reference)
import jax
import jax.numpy as jnp
from jax import lax
from jax.experimental import pallas as pl
from jax.experimental.pallas import tpu as pltpu

N_DEV = 4
SQ = 256
D = 1024
HQ = 8
DH = 128
SKV = 4096
NCLS = 4
QB = 64
KPC = SKV // NCLS
SCALE = 0.08838834764831843


def kernel(x, Wq, K_ext, V_ext, Wo):
    i = lax.axis_index("i")
    xi = x[0].astype(jnp.bfloat16)
    wq = Wq.astype(jnp.bfloat16)
    wo = Wo.astype(jnp.bfloat16).reshape(HQ, DH, D)

    def prep(a):
        a = lax.dynamic_slice_in_dim(a[0], i * HQ, HQ, axis=1)
        a = a.reshape(SKV // QB // NCLS, NCLS, QB, HQ, DH)
        a = a.transpose(1, 3, 0, 2, 4)
        return a.reshape(NCLS, HQ, KPC, DH).astype(jnp.bfloat16)

    kc = prep(K_ext)
    vc = prep(V_ext)

    def body(x_ref, wq_ref, kc_ref, vc_ref, wo_ref, out_ref,
             comm, sendbuf, recvbuf, ag_send, ag_recv, rs_send, rs_recv):
        my = lax.axis_index("i")
        left = lax.rem(my + N_DEV - 1, N_DEV)
        right = lax.rem(my + 1, N_DEV)

        barrier = pltpu.get_barrier_semaphore()
        for nbr in (left, right):
            pl.semaphore_signal(barrier, inc=1, device_id=(nbr,),
                                device_id_type=pl.DeviceIdType.MESH)
        pl.semaphore_wait(barrier, 2)

        for h in range(N_DEV - 1):
            src = x_ref if h == 0 else comm.at[h - 1]
            rdma = pltpu.make_async_remote_copy(
                src_ref=src,
                dst_ref=comm.at[h],
                send_sem=ag_send.at[h],
                recv_sem=ag_recv.at[h],
                device_id=(right,),
                device_id_type=pl.DeviceIdType.MESH,
            )
            rdma.start()
            rdma.wait()

        def block_partial(xb):
            q = lax.dot_general(xb, wq_ref[...], (((1,), (0,)), ((), ())),
                                preferred_element_type=jnp.float32)
            q = q.astype(jnp.bfloat16).reshape(SQ, HQ, DH)
            outs = []
            for c in range(NCLS):
                qc = q[c * QB:(c + 1) * QB]
                s = lax.dot_general(qc, kc_ref[c],
                                    (((2,), (2,)), ((1,), (0,))),
                                    preferred_element_type=jnp.float32)
                s = s * SCALE
                m = jnp.max(s, axis=-1, keepdims=True)
                w = jnp.exp(s - m)
                w = (w / jnp.sum(w, axis=-1, keepdims=True)).astype(jnp.bfloat16)
                ctx = lax.dot_general(w, vc_ref[c],
                                      (((2,), (1,)), ((0,), (0,))),
                                      preferred_element_type=jnp.float32)
                ctx = ctx.astype(jnp.bfloat16)
                po = lax.dot_general(ctx, wo_ref[...],
                                     (((2,), (1,)), ((0,), (0,))),
                                     preferred_element_type=jnp.float32)
                outs.append(jnp.sum(po, axis=0))
            return jnp.concatenate(outs, axis=0)

        own = block_partial(x_ref[...])

        for s in range(N_DEV - 1):
            p = block_partial(comm[s])
            if s > 0:
                p = p + recvbuf[s - 1]
            sendbuf[s, :, :] = p
            rdma = pltpu.make_async_remote_copy(
                src_ref=sendbuf.at[s],
                dst_ref=recvbuf.at[s],
                send_sem=rs_send.at[s],
                recv_sem=rs_recv.at[s],
                device_id=(right,),
                device_id_type=pl.DeviceIdType.MESH,
            )
            rdma.start()
            rdma.wait()

        out_ref[...] = recvbuf[N_DEV - 2] + own

    out = pl.pallas_call(
        body,
        out_shape=jax.ShapeDtypeStruct((SQ, D), jnp.float32),
        in_specs=[pl.BlockSpec(memory_space=pltpu.VMEM)] * 5,
        out_specs=pl.BlockSpec(memory_space=pltpu.VMEM),
        scratch_shapes=[
            pltpu.VMEM((N_DEV - 1, SQ, D), jnp.bfloat16),
            pltpu.VMEM((N_DEV - 1, SQ, D), jnp.float32),
            pltpu.VMEM((N_DEV - 1, SQ, D), jnp.float32),
            pltpu.SemaphoreType.DMA((N_DEV - 1,)),
            pltpu.SemaphoreType.DMA((N_DEV - 1,)),
            pltpu.SemaphoreType.DMA((N_DEV - 1,)),
            pltpu.SemaphoreType.DMA((N_DEV - 1,)),
        ],
        compiler_params=pltpu.CompilerParams(collective_id=0),
    )(xi, wq, kc, vc, wo)
    return out.reshape(1, SQ, D)


# device time: 76772 ns/iter; 1.6004x vs baseline; 1.6004x over previous
import jax
import jax.numpy as jnp
from jax import lax
from jax.experimental import pallas as pl
from jax.experimental.pallas import tpu as pltpu

N_DEV = 4
SQ = 256
D = 1024
HQ = 8
DH = 128
SKV = 4096
NCLS = 4
QB = 64
KPC = SKV // NCLS
SCALE = 0.08838834764831843


def kernel(x, Wq, K_ext, V_ext, Wo):
    i = lax.axis_index("i")
    xi = x[0].astype(jnp.bfloat16)
    wq = Wq.astype(jnp.bfloat16)
    wo = Wo.astype(jnp.bfloat16).reshape(HQ, DH, D)

    def prep(a):
        a = lax.dynamic_slice_in_dim(a[0], i * HQ, HQ, axis=1)
        a = a.reshape(SKV // QB // NCLS, NCLS, QB, HQ, DH)
        a = a.transpose(1, 3, 0, 2, 4)
        return a.reshape(NCLS, HQ, KPC, DH).astype(jnp.bfloat16)

    kc = prep(K_ext)
    vc = prep(V_ext)

    def body(x_ref, wq_ref, kc_ref, vc_ref, wo_ref, out_ref,
             comm, sendbuf, recvbuf, ag_send, ag_recv, rs_send, rs_recv):
        my = lax.axis_index("i")
        left = lax.rem(my + N_DEV - 1, N_DEV)
        right = lax.rem(my + 1, N_DEV)

        barrier = pltpu.get_barrier_semaphore()
        for nbr in (left, right):
            pl.semaphore_signal(barrier, inc=1, device_id=(nbr,),
                                device_id_type=pl.DeviceIdType.MESH)
        pl.semaphore_wait(barrier, 2)

        def ag_rdma(h):
            return pltpu.make_async_remote_copy(
                src_ref=x_ref if h == 0 else comm.at[h - 1],
                dst_ref=comm.at[h],
                send_sem=ag_send.at[h],
                recv_sem=ag_recv.at[h],
                device_id=(right,),
                device_id_type=pl.DeviceIdType.MESH,
            )

        def rs_rdma(s):
            return pltpu.make_async_remote_copy(
                src_ref=sendbuf.at[s],
                dst_ref=recvbuf.at[s],
                send_sem=rs_send.at[s],
                recv_sem=rs_recv.at[s],
                device_id=(right,),
                device_id_type=pl.DeviceIdType.MESH,
            )

        def block_partial(xb):
            q = lax.dot_general(xb, wq_ref[...], (((1,), (0,)), ((), ())),
                                preferred_element_type=jnp.float32)
            q = q.astype(jnp.bfloat16).reshape(SQ, HQ, DH)
            outs = []
            for c in range(NCLS):
                qc = q[c * QB:(c + 1) * QB]
                s = lax.dot_general(qc, kc_ref[c],
                                    (((2,), (2,)), ((1,), (0,))),
                                    preferred_element_type=jnp.float32)
                s = s * SCALE
                m = jnp.max(s, axis=-1, keepdims=True)
                w = jnp.exp(s - m)
                w = (w / jnp.sum(w, axis=-1, keepdims=True)).astype(jnp.bfloat16)
                ctx = lax.dot_general(w, vc_ref[c],
                                      (((2,), (1,)), ((0,), (0,))),
                                      preferred_element_type=jnp.float32)
                ctx = ctx.astype(jnp.bfloat16)
                po = lax.dot_general(ctx, wo_ref[...],
                                     (((2,), (1,)), ((0,), (0,))),
                                     preferred_element_type=jnp.float32)
                outs.append(jnp.sum(po, axis=0))
            return jnp.concatenate(outs, axis=0)

        ag = [ag_rdma(h) for h in range(N_DEV - 1)]
        rs = [rs_rdma(s) for s in range(N_DEV - 1)]

        ag[0].start()
        own = block_partial(x_ref[...])

        ag[0].wait_recv()
        ag[1].start()
        p0 = block_partial(comm[0])
        sendbuf[0, :, :] = p0.astype(jnp.bfloat16)
        rs[0].start()

        ag[1].wait_recv()
        ag[2].start()
        p1 = block_partial(comm[1])
        rs[0].wait_recv()
        sendbuf[1, :, :] = (p1 + recvbuf[0].astype(jnp.float32)).astype(
            jnp.bfloat16)
        rs[1].start()

        ag[2].wait_recv()
        p2 = block_partial(comm[2])
        rs[1].wait_recv()
        sendbuf[2, :, :] = (p2 + recvbuf[1].astype(jnp.float32)).astype(
            jnp.bfloat16)
        rs[2].start()

        rs[2].wait_recv()
        out_ref[...] = own + recvbuf[2].astype(jnp.float32)

        for d in ag + rs:
            d.wait_send()

    out = pl.pallas_call(
        body,
        out_shape=jax.ShapeDtypeStruct((SQ, D), jnp.float32),
        in_specs=[pl.BlockSpec(memory_space=pltpu.VMEM)] * 5,
        out_specs=pl.BlockSpec(memory_space=pltpu.VMEM),
        scratch_shapes=[
            pltpu.VMEM((N_DEV - 1, SQ, D), jnp.bfloat16),
            pltpu.VMEM((N_DEV - 1, SQ, D), jnp.bfloat16),
            pltpu.VMEM((N_DEV - 1, SQ, D), jnp.bfloat16),
            pltpu.SemaphoreType.DMA((N_DEV - 1,)),
            pltpu.SemaphoreType.DMA((N_DEV - 1,)),
            pltpu.SemaphoreType.DMA((N_DEV - 1,)),
            pltpu.SemaphoreType.DMA((N_DEV - 1,)),
        ],
        compiler_params=pltpu.CompilerParams(collective_id=0),
    )(xi, wq, kc, vc, wo)
    return out.reshape(1, SQ, D)
